# baseline (speedup 1.0000x reference)
import functools

import jax
import jax.numpy as jnp
from jax import lax
from jax.experimental import pallas as pl
from jax.experimental.pallas import tpu as pltpu
from jax.experimental.pallas import tpu_sc as plsc

BATCH = 4096
HIST = 200
DIM = 64
LANES = 16
NDREG = DIM // LANES
NROWS = 1000000

SUB = 8
TLANES = 128
NKT = DIM // SUB
RPAD = 1000064
NT = RPAD // TLANES
BLK_WORDS = TLANES * DIM
P1_NBUF = 4

CHUNKS = ((0, 128), (128, 72))
NFULL = HIST // LANES
TAIL = HIST - NFULL * LANES
NBUF = 4


@functools.lru_cache(maxsize=None)
def _make_detile(num_cores, num_subcores):
    nw = num_cores * num_subcores
    nfull = NT - 1
    niter = (nfull + nw - 1) // nw
    mesh = plsc.VectorSubcoreMesh(
        core_axis_name="c", subcore_axis_name="s",
        num_cores=num_cores, num_subcores=num_subcores)

    @functools.partial(
        pl.kernel,
        out_type=jax.ShapeDtypeStruct((NROWS * DIM,), jnp.float32),
        mesh=mesh,
        scratch_types=[
            pltpu.VMEM((P1_NBUF, DIM, TLANES), jnp.float32),
            pltpu.VMEM((P1_NBUF, BLK_WORDS), jnp.float32),
        ] + [pltpu.SemaphoreType.DMA] * P1_NBUF
          + [pltpu.SemaphoreType.DMA] * P1_NBUF,
        compiler_params=pltpu.CompilerParams(
            use_tc_tiling_on_sc=True, needs_layout_passes=False),
    )
    def detile_kernel(tbl_t_hbm, side_hbm, out_hbm, src_v, dst_v, *sems):
        gsems = sems[:P1_NBUF]
        osems = sems[P1_NBUF:]
        wid = lax.axis_index("s") * num_cores + lax.axis_index("c")

        cvec = lax.iota(jnp.int32, LANES)

        def issue_src(t, p):
            pltpu.async_copy(
                tbl_t_hbm.at[pl.ds(0, DIM), pl.ds(t * TLANES, TLANES)],
                src_v.at[p], gsems[p])

        def drain_src(t, p):
            pltpu.make_async_copy(
                tbl_t_hbm.at[pl.ds(0, DIM), pl.ds(t * TLANES, TLANES)],
                src_v.at[p], gsems[p]).wait()

        def transpose(p, nrr):
            pvec = jnp.full((LANES,), p, jnp.int32)
            unroll = 4

            def body(r4, carry):
                for u in range(unroll):
                    rr = r4 * unroll + u
                    rrv = cvec * 0 + rr
                    for m in range(DIM // LANES):
                        vals = plsc.load_gather(
                            src_v, [pvec, LANES * m + cvec, rrv])
                        dst_v[p, pl.ds(rr * DIM + LANES * m, LANES)] = vals
                return carry
            lax.fori_loop(0, nrr // unroll, body, 0)

        def issue_out(t, p, words):
            pltpu.async_copy(dst_v.at[p, pl.ds(0, words)],
                             out_hbm.at[pl.ds(t * BLK_WORDS, words)],
                             osems[p])

        def drain_out(t, p, words):
            pltpu.make_async_copy(
                dst_v.at[p, pl.ds(0, words)],
                out_hbm.at[pl.ds(t * BLK_WORDS, words)],
                osems[p]).wait()

        for p in range(P1_NBUF):
            issue_src(wid + nw * p, p)

        def outer(g, carry):
            for p in range(P1_NBUF):
                j = g * P1_NBUF + p
                t = wid + nw * j

                @pl.when(t < nfull)
                def _():
                    drain_src(t, p)

                    @pl.when(j >= P1_NBUF)
                    def _():
                        drain_out(wid + nw * (j - P1_NBUF), p, BLK_WORDS)

                    transpose(p, TLANES)
                    issue_out(t, p, BLK_WORDS)
                    tn = wid + nw * (j + P1_NBUF)

                    @pl.when(tn < nfull)
                    def _():
                        issue_src(tn, p)
            return carry

        nouter = (niter + P1_NBUF - 1) // P1_NBUF
        lax.fori_loop(0, nouter, outer, 0)

        for p in range(P1_NBUF):
            drain_out(0, p, BLK_WORDS)

        nlast = NROWS - nfull * TLANES

        @pl.when(wid == 4)
        def _():
            pltpu.sync_copy(side_hbm, dst_v.at[0, pl.ds(0, nlast * DIM)])
            issue_out(nfull, 0, nlast * DIM)
            drain_out(nfull, 0, nlast * DIM)

    return detile_kernel


@functools.lru_cache(maxsize=None)
def _make_gather(num_cores, num_subcores):
    nw = num_cores * num_subcores
    bpw = BATCH // nw
    mesh = plsc.VectorSubcoreMesh(
        core_axis_name="c", subcore_axis_name="s",
        num_cores=num_cores, num_subcores=num_subcores)

    @functools.partial(
        pl.kernel,
        out_type=jax.ShapeDtypeStruct((BATCH, DIM), jnp.float32),
        mesh=mesh,
        scratch_types=[
            pltpu.VMEM((bpw, HIST), jnp.int32),
            pltpu.VMEM((bpw, HIST), jnp.float32),
            pltpu.VMEM((NBUF, HIST, DIM), jnp.float32),
            pltpu.VMEM((bpw, DIM), jnp.float32),
        ] + [pltpu.SemaphoreType.DMA] * NBUF,
        compiler_params=pltpu.CompilerParams(use_tc_tiling_on_sc=False),
    )
    def emb_kernel(x_hbm, w_hbm, table_hbm, out_hbm, idx_v, w_v, rows_v,
                   out_v, *sems):
        wid = lax.axis_index("s") * num_cores + lax.axis_index("c")
        base = wid * bpw
        pltpu.sync_copy(x_hbm.at[pl.ds(base, bpw)], idx_v)
        pltpu.sync_copy(w_hbm.at[pl.ds(base, bpw)], w_v)

        def issue(b, p):
            for off, sz in CHUNKS:
                pltpu.async_copy(
                    table_hbm.at[idx_v.at[b, pl.ds(off, sz)]],
                    rows_v.at[p, pl.ds(off, sz)], sems[p])

        def drain(b, p):
            for off, sz in CHUNKS:
                pltpu.make_async_copy(
                    table_hbm.at[idx_v.at[b, pl.ds(off, sz)]],
                    rows_v.at[p, pl.ds(off, sz)], sems[p]).wait()

        for p in range(NBUF):
            issue(p, p)

        def outer(g, carry):
            for p in range(NBUF):
                b = g * NBUF + p
                drain(b, p)

                def accumulate(gbase, ks, acc):
                    wv = w_v[b, pl.ds(gbase, LANES)]
                    for k in ks:
                        wl = wv[k]
                        acc = tuple(
                            acc[d] + wl * rows_v[p, gbase + k,
                                                 pl.ds(LANES * d, LANES)]
                            for d in range(NDREG))
                    return acc

                def inner(gg, acc):
                    return accumulate(LANES * gg, range(LANES), acc)

                acc = lax.fori_loop(
                    0, NFULL, inner,
                    tuple(jnp.zeros((LANES,), jnp.float32)
                          for _ in range(NDREG)))
                acc = accumulate(HIST - LANES, range(LANES - TAIL, LANES),
                                 acc)
                for d in range(NDREG):
                    out_v[b, pl.ds(LANES * d, LANES)] = acc[d]

                @pl.when(b + NBUF < bpw)
                def _():
                    issue(b + NBUF, p)
            return carry

        lax.fori_loop(0, bpw // NBUF, outer, 0)
        pltpu.sync_copy(out_v, out_hbm.at[pl.ds(base, bpw)])

    return emb_kernel


def kernel(x, w, weight):
    try:
        info = plsc.get_sparse_core_info()
        nc, ns = info.num_cores, info.num_subcores
    except Exception:
        nc, ns = 2, 16
    tail = NROWS - (NT - 1) * TLANES
    tbl_lin = _make_detile(nc, ns)(
        jnp.swapaxes(weight, 0, 1), weight[NROWS - tail:].reshape(-1))
    return _make_gather(nc, ns)(
        x.astype(jnp.int32), w, tbl_lin.reshape(NROWS, DIM))


# speedup vs baseline: 2.4730x; 2.4730x over previous
import functools

import jax
import jax.numpy as jnp
from jax import lax
from jax.experimental import pallas as pl
from jax.experimental.pallas import tpu as pltpu
from jax.experimental.pallas import tpu_sc as plsc

BATCH = 4096
HIST = 200
DIM = 64
LANES = 16
NDREG = DIM // LANES
NROWS = 1000000

SUB = 8
TLANES = 128
NKT = DIM // SUB
RPAD = 1000064
NT = RPAD // TLANES
BLK_WORDS = TLANES * DIM
P1_NBUF = 4

CHUNKS = ((0, 128), (128, 72))
NFULL = HIST // LANES
TAIL = HIST - NFULL * LANES
NBUF = 4


@functools.lru_cache(maxsize=None)
def _make_detile(num_cores, num_subcores):
    nw = num_cores * num_subcores
    nfull = NT - 1
    niter = (nfull + nw - 1) // nw
    mesh = plsc.VectorSubcoreMesh(
        core_axis_name="c", subcore_axis_name="s",
        num_cores=num_cores, num_subcores=num_subcores)

    @functools.partial(
        pl.kernel,
        out_type=jax.ShapeDtypeStruct((NROWS * DIM,), jnp.float32),
        mesh=mesh,
        scratch_types=[
            pltpu.VMEM((P1_NBUF, DIM, TLANES), jnp.float32),
            pltpu.VMEM((P1_NBUF, BLK_WORDS), jnp.float32),
        ] + [pltpu.SemaphoreType.DMA] * P1_NBUF
          + [pltpu.SemaphoreType.DMA] * P1_NBUF,
        compiler_params=pltpu.CompilerParams(
            use_tc_tiling_on_sc=True, needs_layout_passes=False),
    )
    def detile_kernel(tbl_t_hbm, side_hbm, out_hbm, src_v, dst_v, *sems):
        gsems = sems[:P1_NBUF]
        osems = sems[P1_NBUF:]
        wid = lax.axis_index("s") * num_cores + lax.axis_index("c")

        cvec = lax.iota(jnp.int32, LANES)

        def issue_src(t, p):
            pltpu.async_copy(
                tbl_t_hbm.at[pl.ds(0, DIM), pl.ds(t * TLANES, TLANES)],
                src_v.at[p], gsems[p])

        def drain_src(t, p):
            pltpu.make_async_copy(
                tbl_t_hbm.at[pl.ds(0, DIM), pl.ds(t * TLANES, TLANES)],
                src_v.at[p], gsems[p]).wait()

        def transpose(p, nrr):
            pvec = jnp.full((LANES,), p, jnp.int32)

            def body(am, carry):
                r0 = lax.shift_right_logical(am, 2) * LANES
                c0 = (am & 3) * LANES
                for d in range(LANES):
                    cd = (cvec + d) & (LANES - 1)
                    vals = plsc.load_gather(
                        src_v, [pvec, c0 + cvec, r0 + cd])
                    plsc.store_scatter(
                        dst_v,
                        [pvec, r0 * DIM + cd * DIM + c0 + cvec],
                        vals)
                return carry
            lax.fori_loop(0, (nrr // LANES) * (DIM // LANES), body, 0)

        def issue_out(t, p, words):
            pltpu.async_copy(dst_v.at[p, pl.ds(0, words)],
                             out_hbm.at[pl.ds(t * BLK_WORDS, words)],
                             osems[p])

        def drain_out(t, p, words):
            pltpu.make_async_copy(
                dst_v.at[p, pl.ds(0, words)],
                out_hbm.at[pl.ds(t * BLK_WORDS, words)],
                osems[p]).wait()

        for p in range(P1_NBUF):
            issue_src(wid + nw * p, p)

        def outer(g, carry):
            for p in range(P1_NBUF):
                j = g * P1_NBUF + p
                t = wid + nw * j

                @pl.when(t < nfull)
                def _():
                    drain_src(t, p)

                    @pl.when(j >= P1_NBUF)
                    def _():
                        drain_out(wid + nw * (j - P1_NBUF), p, BLK_WORDS)

                    transpose(p, TLANES)
                    issue_out(t, p, BLK_WORDS)
                    tn = wid + nw * (j + P1_NBUF)

                    @pl.when(tn < nfull)
                    def _():
                        issue_src(tn, p)
            return carry

        nouter = (niter + P1_NBUF - 1) // P1_NBUF
        lax.fori_loop(0, nouter, outer, 0)

        for p in range(P1_NBUF):
            drain_out(0, p, BLK_WORDS)

        nlast = NROWS - nfull * TLANES

        @pl.when(wid == 4)
        def _():
            pltpu.sync_copy(side_hbm, dst_v.at[0, pl.ds(0, nlast * DIM)])
            issue_out(nfull, 0, nlast * DIM)
            drain_out(nfull, 0, nlast * DIM)

    return detile_kernel


@functools.lru_cache(maxsize=None)
def _make_gather(num_cores, num_subcores):
    nw = num_cores * num_subcores
    bpw = BATCH // nw
    mesh = plsc.VectorSubcoreMesh(
        core_axis_name="c", subcore_axis_name="s",
        num_cores=num_cores, num_subcores=num_subcores)

    @functools.partial(
        pl.kernel,
        out_type=jax.ShapeDtypeStruct((BATCH, DIM), jnp.float32),
        mesh=mesh,
        scratch_types=[
            pltpu.VMEM((bpw, HIST), jnp.int32),
            pltpu.VMEM((bpw, HIST), jnp.float32),
            pltpu.VMEM((NBUF, HIST, DIM), jnp.float32),
            pltpu.VMEM((bpw, DIM), jnp.float32),
        ] + [pltpu.SemaphoreType.DMA] * NBUF,
        compiler_params=pltpu.CompilerParams(use_tc_tiling_on_sc=False),
    )
    def emb_kernel(x_hbm, w_hbm, table_hbm, out_hbm, idx_v, w_v, rows_v,
                   out_v, *sems):
        wid = lax.axis_index("s") * num_cores + lax.axis_index("c")
        base = wid * bpw
        pltpu.sync_copy(x_hbm.at[pl.ds(base, bpw)], idx_v)
        pltpu.sync_copy(w_hbm.at[pl.ds(base, bpw)], w_v)

        def issue(b, p):
            for off, sz in CHUNKS:
                pltpu.async_copy(
                    table_hbm.at[idx_v.at[b, pl.ds(off, sz)]],
                    rows_v.at[p, pl.ds(off, sz)], sems[p])

        def drain(b, p):
            for off, sz in CHUNKS:
                pltpu.make_async_copy(
                    table_hbm.at[idx_v.at[b, pl.ds(off, sz)]],
                    rows_v.at[p, pl.ds(off, sz)], sems[p]).wait()

        for p in range(NBUF):
            issue(p, p)

        def outer(g, carry):
            for p in range(NBUF):
                b = g * NBUF + p
                drain(b, p)

                def accumulate(gbase, ks, acc):
                    wv = w_v[b, pl.ds(gbase, LANES)]
                    for k in ks:
                        wl = wv[k]
                        acc = tuple(
                            acc[d] + wl * rows_v[p, gbase + k,
                                                 pl.ds(LANES * d, LANES)]
                            for d in range(NDREG))
                    return acc

                def inner(gg, acc):
                    return accumulate(LANES * gg, range(LANES), acc)

                acc = lax.fori_loop(
                    0, NFULL, inner,
                    tuple(jnp.zeros((LANES,), jnp.float32)
                          for _ in range(NDREG)))
                acc = accumulate(HIST - LANES, range(LANES - TAIL, LANES),
                                 acc)
                for d in range(NDREG):
                    out_v[b, pl.ds(LANES * d, LANES)] = acc[d]

                @pl.when(b + NBUF < bpw)
                def _():
                    issue(b + NBUF, p)
            return carry

        lax.fori_loop(0, bpw // NBUF, outer, 0)
        pltpu.sync_copy(out_v, out_hbm.at[pl.ds(base, bpw)])

    return emb_kernel


def kernel(x, w, weight):
    try:
        info = plsc.get_sparse_core_info()
        nc, ns = info.num_cores, info.num_subcores
    except Exception:
        nc, ns = 2, 16
    tail = NROWS - (NT - 1) * TLANES
    tbl_lin = _make_detile(nc, ns)(
        jnp.swapaxes(weight, 0, 1), weight[NROWS - tail:].reshape(-1))
    return _make_gather(nc, ns)(
        x.astype(jnp.int32), w, tbl_lin.reshape(NROWS, DIM))
